# Initial kernel scaffold; baseline (speedup 1.0000x reference)
import functools

import jax
import jax.numpy as jnp
from jax import lax
from jax.experimental import pallas as pl
from jax.experimental.pallas import tpu as pltpu
from jax.experimental.pallas import tpu_sc as plsc

S = 10000
D = 128
NW = 32
SPW = 313
LAST = S - (NW - 1) * SPW
C = 512
NVEC = D // 16


def _make_kernel(N):
    mesh = plsc.VectorSubcoreMesh(core_axis_name="c", subcore_axis_name="s")

    @functools.partial(
        pl.kernel,
        out_type=jax.ShapeDtypeStruct((S, D), jnp.float32),
        mesh=mesh,
        scratch_types=[
            pltpu.VMEM((C, D), jnp.float32),
            pltpu.VMEM((C,), jnp.int32),
            pltpu.VMEM((SPW, D), jnp.float32),
            pltpu.VMEM((48,), jnp.int32),
        ],
    )
    def seg_max(feats_hbm, batch_hbm, offs_hbm, out_hbm, fbuf, ibuf, table, offs_v):
        w = lax.axis_index("s") * 2 + lax.axis_index("c")
        seg_base = w * SPW

        zero = jnp.zeros((16,), jnp.float32)

        def zbody(i, _):
            for j in range(NVEC):
                table[i, pl.ds(j * 16, 16)] = zero
            return 0

        lax.fori_loop(0, SPW, zbody, 0)

        pltpu.sync_copy(offs_hbm, offs_v)
        r0 = offs_v[w]
        r1 = offs_v[w + 1]
        r0a = (r0 // 8) * 8
        nchunks = (r1 - r0a + C - 1) // C

        def row_body(i, carry):
            cur_seg = carry[0]
            accs = carry[1:]
            s = ibuf[i]
            row = [fbuf[i, pl.ds(j * 16, 16)] for j in range(NVEC)]
            flush = s != cur_seg

            @pl.when(flush & (cur_seg >= 0))
            def _():
                sl = cur_seg - seg_base
                for j in range(NVEC):
                    table[sl, pl.ds(j * 16, 16)] = accs[j]

            new_accs = tuple(
                jnp.where(flush, row[j], jnp.maximum(accs[j], row[j]))
                for j in range(NVEC)
            )
            return (s,) + new_accs

        def chunk_body(c, carry):
            base_u = r0a + c * C
            base = jnp.minimum(base_u, N - C)
            pltpu.sync_copy(feats_hbm.at[pl.ds(base, C)], fbuf)
            pltpu.sync_copy(batch_hbm.at[pl.ds(base, C)], ibuf)
            i0 = jnp.maximum(r0, base_u) - base
            i1 = jnp.minimum(r1, base_u + C) - base
            return lax.fori_loop(i0, i1, row_body, carry)

        init = (jnp.int32(-1),) + tuple(
            jnp.full((16,), -jnp.inf, jnp.float32) for _ in range(NVEC)
        )
        final = lax.fori_loop(0, nchunks, chunk_body, init)
        cur_seg = final[0]

        @pl.when(cur_seg >= 0)
        def _():
            sl = cur_seg - seg_base
            for j in range(NVEC):
                table[sl, pl.ds(j * 16, 16)] = final[1 + j]

        @pl.when(w < NW - 1)
        def _():
            pltpu.sync_copy(table, out_hbm.at[pl.ds(seg_base, SPW)])

        @pl.when(w == NW - 1)
        def _():
            pltpu.sync_copy(table.at[pl.ds(0, LAST)], out_hbm.at[pl.ds(seg_base, LAST)])

    return seg_max


def kernel(feats, batch):
    N = feats.shape[0]
    bounds = jnp.minimum(jnp.arange(NW + 1, dtype=jnp.int32) * SPW, S)
    offs = jnp.searchsorted(batch, bounds, side="left").astype(jnp.int32)
    offs = jnp.concatenate([offs, jnp.zeros((48 - NW - 1,), jnp.int32)])
    return _make_kernel(N)(feats, batch, offs)


# speedup vs baseline: 3.0214x; 3.0214x over previous
import functools

import jax
import jax.numpy as jnp
from jax import lax
from jax.experimental import pallas as pl
from jax.experimental.pallas import tpu as pltpu
from jax.experimental.pallas import tpu_sc as plsc

S = 10000
D = 128
NW = 32
SPW = 312
LAST = S - (NW - 1) * SPW
C = 512
NVEC = D // 16


def _make_kernel(N):
    mesh = plsc.VectorSubcoreMesh(core_axis_name="c", subcore_axis_name="s")

    @functools.partial(
        pl.kernel,
        out_type=jax.ShapeDtypeStruct((S * D,), jnp.float32),
        mesh=mesh,
        scratch_types=[
            pltpu.VMEM((C * D,), jnp.float32),
            pltpu.VMEM((C + 16,), jnp.int32),
            pltpu.VMEM((LAST * D,), jnp.float32),
            pltpu.VMEM((48,), jnp.int32),
        ],
    )
    def seg_max(feats_hbm, batch_hbm, offs_hbm, out_hbm, fbuf, ibuf, table, offs_v):
        w = lax.axis_index("s") * 2 + lax.axis_index("c")
        seg_base = w * SPW

        zero = jnp.zeros((16,), jnp.float32)

        def zbody(i, _):
            table[pl.ds(i * 16, 16)] = zero
            return 0

        lax.fori_loop(0, LAST * D // 16, zbody, 0)

        pltpu.sync_copy(offs_hbm, offs_v)
        r0 = offs_v[pl.ds(w, 16)][0]
        r1 = offs_v[pl.ds(w + 1, 16)][0]
        r0a = (r0 // 8) * 8
        nchunks = (r1 - r0a + C - 1) // C

        def row_body(i, carry):
            cur_seg = carry[0]
            accs = carry[1:]
            s = ibuf[pl.ds(i, 16)][0]
            row = [fbuf[pl.ds(i * D + j * 16, 16)] for j in range(NVEC)]
            flush = s != cur_seg

            @pl.when(flush & (cur_seg >= 0))
            def _():
                off = (cur_seg - seg_base) * D
                for j in range(NVEC):
                    table[pl.ds(off + j * 16, 16)] = accs[j]

            new_accs = tuple(
                jnp.where(flush, row[j], jnp.maximum(accs[j], row[j]))
                for j in range(NVEC)
            )
            return (s,) + new_accs

        def chunk_body(c, carry):
            base_u = r0a + c * C
            base = jnp.minimum(base_u, N - C)
            pltpu.sync_copy(feats_hbm.at[pl.ds(base * D, C * D)], fbuf)
            pltpu.sync_copy(batch_hbm.at[pl.ds(base, C)], ibuf.at[pl.ds(0, C)])
            i0 = jnp.maximum(r0, base_u) - base
            i1 = jnp.minimum(r1, base_u + C) - base
            return lax.fori_loop(i0, i1, row_body, carry)

        init = (jnp.int32(-1),) + tuple(
            jnp.full((16,), -jnp.inf, jnp.float32) for _ in range(NVEC)
        )
        final = lax.fori_loop(0, nchunks, chunk_body, init)
        cur_seg = final[0]

        @pl.when(cur_seg >= 0)
        def _():
            off = (cur_seg - seg_base) * D
            for j in range(NVEC):
                table[pl.ds(off + j * 16, 16)] = final[1 + j]

        @pl.when(w < NW - 1)
        def _():
            pltpu.sync_copy(table.at[pl.ds(0, SPW * D)],
                            out_hbm.at[pl.ds(seg_base * D, SPW * D)])

        @pl.when(w == NW - 1)
        def _():
            pltpu.sync_copy(table, out_hbm.at[pl.ds(seg_base * D, LAST * D)])

    return seg_max


def kernel(feats, batch):
    N = feats.shape[0]
    bounds = jnp.concatenate(
        [jnp.arange(NW, dtype=jnp.int32) * SPW, jnp.full((1,), S, jnp.int32)]
    )
    offs = jnp.searchsorted(batch, bounds, side="left").astype(jnp.int32)
    offs = jnp.concatenate([offs, jnp.zeros((48 - NW - 1,), jnp.int32)])
    out = _make_kernel(N)(feats.reshape(-1), batch, offs)
    return out.reshape(S, D)


# speedup vs baseline: 3.7686x; 1.2473x over previous
import functools

import jax
import jax.numpy as jnp
from jax import lax
from jax.experimental import pallas as pl
from jax.experimental.pallas import tpu as pltpu
from jax.experimental.pallas import tpu_sc as plsc

S = 10000
D = 128
NW = 32
SPW = 312
LAST = S - (NW - 1) * SPW
C = 320
NVEC = D // 16


def _make_kernel(N):
    mesh = plsc.VectorSubcoreMesh(core_axis_name="c", subcore_axis_name="s")

    @functools.partial(
        pl.kernel,
        out_type=jax.ShapeDtypeStruct((S * D,), jnp.float32),
        mesh=mesh,
        scratch_types=[
            pltpu.VMEM((C * D,), jnp.float32),
            pltpu.VMEM((C * D,), jnp.float32),
            pltpu.VMEM((C + 16,), jnp.int32),
            pltpu.VMEM((C + 16,), jnp.int32),
            pltpu.VMEM((LAST * D,), jnp.float32),
            pltpu.VMEM((48,), jnp.int32),
            pltpu.SemaphoreType.DMA,
            pltpu.SemaphoreType.DMA,
            pltpu.SemaphoreType.DMA,
            pltpu.SemaphoreType.DMA,
        ],
    )
    def seg_max(feats_hbm, batch_hbm, offs_hbm, out_hbm,
                fbuf0, fbuf1, ibuf0, ibuf1, table, offs_v,
                semf0, semf1, semi0, semi1):
        w = lax.axis_index("s") * 2 + lax.axis_index("c")
        seg_base = w * SPW

        zero = jnp.zeros((16,), jnp.float32)

        def zbody(i, _):
            table[pl.ds(i * 16, 16)] = zero
            return 0

        lax.fori_loop(0, LAST * D // 16, zbody, 0)

        pltpu.sync_copy(offs_hbm, offs_v)
        r0 = offs_v[pl.ds(w, 16)][0]
        r1 = offs_v[pl.ds(w + 1, 16)][0]
        r0a = (r0 // 8) * 8
        nchunks = (r1 - r0a + C - 1) // C

        def chunk_base(c):
            return jnp.minimum(r0a + c * C, N - C)

        def start(c, fb, ib, semf, semi):
            base = chunk_base(c)
            pltpu.async_copy(feats_hbm.at[pl.ds(base * D, C * D)], fb, semf)
            pltpu.async_copy(batch_hbm.at[pl.ds(base, C)], ib.at[pl.ds(0, C)], semi)

        def drain(fb, ib, semf, semi):
            pltpu.make_async_copy(feats_hbm.at[pl.ds(0, C * D)], fb, semf).wait()
            pltpu.make_async_copy(batch_hbm.at[pl.ds(0, C)], ib.at[pl.ds(0, C)], semi).wait()

        def process(c, fb, ib, carry):
            base_u = r0a + c * C
            base = chunk_base(c)
            i0 = jnp.maximum(r0, jnp.minimum(base_u, r1)) - base
            i1 = jnp.minimum(r1, base_u + C) - base

            def row_body(i, carry):
                cur_seg = carry[0]
                accs = carry[1:]
                s = ib[pl.ds(i, 16)][0]
                row = [fb[pl.ds(i * D + j * 16, 16)] for j in range(NVEC)]
                flush = s != cur_seg

                @pl.when(flush & (cur_seg >= 0))
                def _():
                    off = (cur_seg - seg_base) * D
                    for j in range(NVEC):
                        table[pl.ds(off + j * 16, 16)] = accs[j]

                new_accs = tuple(
                    jnp.where(flush, row[j], jnp.maximum(accs[j], row[j]))
                    for j in range(NVEC)
                )
                return (s,) + new_accs

            return lax.fori_loop(i0, i1, row_body, carry)

        start(jnp.int32(0), fbuf0, ibuf0, semf0, semi0)

        def pair_body(g, carry):
            c0 = 2 * g
            start(c0 + 1, fbuf1, ibuf1, semf1, semi1)
            drain(fbuf0, ibuf0, semf0, semi0)
            carry = process(c0, fbuf0, ibuf0, carry)
            start(c0 + 2, fbuf0, ibuf0, semf0, semi0)
            drain(fbuf1, ibuf1, semf1, semi1)
            return process(c0 + 1, fbuf1, ibuf1, carry)

        init = (jnp.int32(-1),) + tuple(
            jnp.full((16,), -jnp.inf, jnp.float32) for _ in range(NVEC)
        )
        npairs = (nchunks + 1) // 2
        final = lax.fori_loop(0, npairs, pair_body, init)
        drain(fbuf0, ibuf0, semf0, semi0)
        cur_seg = final[0]

        @pl.when(cur_seg >= 0)
        def _():
            off = (cur_seg - seg_base) * D
            for j in range(NVEC):
                table[pl.ds(off + j * 16, 16)] = final[1 + j]

        @pl.when(w < NW - 1)
        def _():
            pltpu.sync_copy(table.at[pl.ds(0, SPW * D)],
                            out_hbm.at[pl.ds(seg_base * D, SPW * D)])

        @pl.when(w == NW - 1)
        def _():
            pltpu.sync_copy(table, out_hbm.at[pl.ds(seg_base * D, LAST * D)])

    return seg_max


def kernel(feats, batch):
    N = feats.shape[0]
    bounds = jnp.concatenate(
        [jnp.arange(NW, dtype=jnp.int32) * SPW, jnp.full((1,), S, jnp.int32)]
    )
    offs = jnp.searchsorted(batch, bounds, side="left").astype(jnp.int32)
    offs = jnp.concatenate([offs, jnp.zeros((48 - NW - 1,), jnp.int32)])
    out = _make_kernel(N)(feats.reshape(-1), batch, offs)
    return out.reshape(S, D)


# speedup vs baseline: 5.2197x; 1.3850x over previous
import functools

import jax
import jax.numpy as jnp
from jax import lax
from jax.experimental import pallas as pl
from jax.experimental.pallas import tpu as pltpu
from jax.experimental.pallas import tpu_sc as plsc

_GDN = lax.GatherDimensionNumbers(
    offset_dims=(), collapsed_slice_dims=(0,), start_index_map=(0,)
)


def _bcast0(v):
    return lax.gather(v, jnp.zeros((16, 1), jnp.int32), _GDN, (1,),
                      mode=lax.GatherScatterMode.PROMISE_IN_BOUNDS)


S = 10000
D = 128
NW = 32
SPW = 312
LAST = S - (NW - 1) * SPW
C = 320
NVEC = D // 16


def _make_kernel(N):
    mesh = plsc.VectorSubcoreMesh(core_axis_name="c", subcore_axis_name="s")

    @functools.partial(
        pl.kernel,
        out_type=jax.ShapeDtypeStruct((S, D), jnp.float32),
        mesh=mesh,
        scratch_types=[
            pltpu.VMEM((C * D,), jnp.float32),
            pltpu.VMEM((C * D,), jnp.float32),
            pltpu.VMEM((C + 16,), jnp.int32),
            pltpu.VMEM((C + 16,), jnp.int32),
            pltpu.VMEM((LAST + 1, D), jnp.float32),
            pltpu.VMEM((48,), jnp.int32),
            pltpu.SemaphoreType.DMA,
            pltpu.SemaphoreType.DMA,
            pltpu.SemaphoreType.DMA,
            pltpu.SemaphoreType.DMA,
        ],
        compiler_params=pltpu.CompilerParams(needs_layout_passes=False),
    )
    def seg_max(feats_hbm, batch_hbm, offs_hbm, out_hbm,
                fbuf0, fbuf1, ibuf0, ibuf1, table, offs_v,
                semf0, semf1, semi0, semi1):
        w = lax.axis_index("s") * 2 + lax.axis_index("c")
        seg_base = w * SPW

        zero = jnp.zeros((16,), jnp.float32)
        iota16 = lax.iota(jnp.int32, 16)
        iotas = [iota16 + j * 16 for j in range(NVEC)]
        segb_vec = jnp.full((16,), seg_base, jnp.int32)

        def zbody(i, _):
            zi = jnp.full((16,), 0, jnp.int32) + i
            for j in range(NVEC):
                plsc.store_scatter(table, [zi, iotas[j]], zero)
            return 0

        lax.fori_loop(0, LAST, zbody, 0)

        pltpu.sync_copy(offs_hbm, offs_v)
        r0 = offs_v[pl.ds(w, 16)][0]
        r1 = offs_v[pl.ds(w + 1, 16)][0]
        r0a = (r0 // 8) * 8
        nchunks = (r1 - r0a + C - 1) // C

        def chunk_base(c):
            return jnp.minimum(r0a + c * C, N - C)

        def start(c, fb, ib, semf, semi):
            base = chunk_base(c)
            pltpu.async_copy(feats_hbm.at[pl.ds(base * D, C * D)], fb, semf)
            pltpu.async_copy(batch_hbm.at[pl.ds(base, C)], ib.at[pl.ds(0, C)], semi)

        def drain(fb, ib, semf, semi):
            pltpu.make_async_copy(feats_hbm.at[pl.ds(0, C * D)], fb, semf).wait()
            pltpu.make_async_copy(batch_hbm.at[pl.ds(0, C)], ib.at[pl.ds(0, C)], semi).wait()

        def process(c, fb, ib, carry):
            base_u = r0a + c * C
            base = chunk_base(c)
            i0 = jnp.maximum(r0, jnp.minimum(base_u, r1)) - base
            i1 = jnp.minimum(r1, base_u + C) - base

            def ids_at(i):
                cur = _bcast0(ib[pl.ds(i, 16)])
                return cur, cur - segb_vec

            def step(i, prev, prow, cur, accs):
                row = [fb[pl.ds(i * D + j * 16, 16)] for j in range(NVEC)]
                bound = cur != prev
                for j in range(NVEC):
                    plsc.store_scatter(table, [prow, iotas[j]], accs[j],
                                       mask=bound)
                new_accs = tuple(
                    jnp.where(bound, row[j], jnp.maximum(accs[j], row[j]))
                    for j in range(NVEC)
                )
                return new_accs

            cur0, row0 = ids_at(i0)

            def body(g, lc):
                prev, prow, cur_a, row_a = lc[0], lc[1], lc[2], lc[3]
                accs = lc[4:]
                a = i0 + 2 * g
                b = jnp.minimum(a + 1, i1 - 1)
                accs = step(a, prev, prow, cur_a, accs)
                cur_b, row_b = ids_at(b)
                accs = step(b, cur_a, row_a, cur_b, accs)
                cur_n, row_n = ids_at(jnp.minimum(a + 2, i1 - 1))
                return (cur_b, row_b, cur_n, row_n) + accs

            out = lax.fori_loop(0, (i1 - i0 + 1) // 2, body,
                                (carry[0], carry[1], cur0, row0)
                                + tuple(carry[2:]))
            nonempty = i0 < i1
            nprev = jnp.where(nonempty, out[0], carry[0])
            nprow = jnp.where(nonempty, out[1], carry[1])
            return (nprev, nprow) + tuple(out[4:])

        start(jnp.int32(0), fbuf0, ibuf0, semf0, semi0)

        def pair_body(g, carry):
            c0 = 2 * g
            start(c0 + 1, fbuf1, ibuf1, semf1, semi1)
            drain(fbuf0, ibuf0, semf0, semi0)
            carry = process(c0, fbuf0, ibuf0, carry)
            start(c0 + 2, fbuf0, ibuf0, semf0, semi0)
            drain(fbuf1, ibuf1, semf1, semi1)
            return process(c0 + 1, fbuf1, ibuf1, carry)

        init = (jnp.full((16,), -1, jnp.int32),
                jnp.full((16,), LAST, jnp.int32)) + tuple(
            jnp.full((16,), -jnp.inf, jnp.float32) for _ in range(NVEC)
        )
        npairs = (nchunks + 1) // 2
        final = lax.fori_loop(0, npairs, pair_body, init)
        drain(fbuf0, ibuf0, semf0, semi0)
        for j in range(NVEC):
            plsc.store_scatter(table, [final[1], iotas[j]], final[2 + j])

        @pl.when(w < NW - 1)
        def _():
            pltpu.sync_copy(table.at[pl.ds(0, SPW)],
                            out_hbm.at[pl.ds(seg_base, SPW)])

        @pl.when(w == NW - 1)
        def _():
            pltpu.sync_copy(table.at[pl.ds(0, LAST)],
                            out_hbm.at[pl.ds(seg_base, LAST)])

    return seg_max


def kernel(feats, batch):
    N = feats.shape[0]
    bounds = jnp.concatenate(
        [jnp.arange(NW, dtype=jnp.int32) * SPW, jnp.full((1,), S, jnp.int32)]
    )
    offs = jnp.searchsorted(batch, bounds, side="left").astype(jnp.int32)
    offs = jnp.concatenate([offs, jnp.zeros((48 - NW - 1,), jnp.int32)])
    return _make_kernel(N)(feats.reshape(-1), batch, offs)


# speedup vs baseline: 5.3782x; 1.0303x over previous
import functools

import jax
import jax.numpy as jnp
from jax import lax
from jax.experimental import pallas as pl
from jax.experimental.pallas import tpu as pltpu
from jax.experimental.pallas import tpu_sc as plsc

_GDN = lax.GatherDimensionNumbers(
    offset_dims=(), collapsed_slice_dims=(0,), start_index_map=(0,)
)


def _bcast0(v):
    return lax.gather(v, jnp.zeros((16, 1), jnp.int32), _GDN, (1,),
                      mode=lax.GatherScatterMode.PROMISE_IN_BOUNDS)


S = 10000
D = 128
NW = 32
SPW = 312
LAST = S - (NW - 1) * SPW
C = 320
NVEC = D // 16


def _make_kernel(N):
    mesh = plsc.VectorSubcoreMesh(core_axis_name="c", subcore_axis_name="s")

    @functools.partial(
        pl.kernel,
        out_type=jax.ShapeDtypeStruct((S, D), jnp.float32),
        mesh=mesh,
        scratch_types=[
            pltpu.VMEM((C * D,), jnp.float32),
            pltpu.VMEM((C * D,), jnp.float32),
            pltpu.VMEM((C + 16,), jnp.int32),
            pltpu.VMEM((C + 16,), jnp.int32),
            pltpu.VMEM((LAST + 1, D), jnp.float32),
            pltpu.VMEM((48,), jnp.int32),
            pltpu.SemaphoreType.DMA,
            pltpu.SemaphoreType.DMA,
            pltpu.SemaphoreType.DMA,
            pltpu.SemaphoreType.DMA,
        ],
        compiler_params=pltpu.CompilerParams(needs_layout_passes=False),
    )
    def seg_max(feats_hbm, batch_hbm, offs_hbm, out_hbm,
                fbuf0, fbuf1, ibuf0, ibuf1, table, offs_v,
                semf0, semf1, semi0, semi1):
        w = lax.axis_index("s") * 2 + lax.axis_index("c")
        seg_base = w * SPW

        zero = jnp.zeros((16,), jnp.float32)
        iota16 = lax.iota(jnp.int32, 16)
        iotas = [iota16 + j * 16 for j in range(NVEC)]
        segb_vec = jnp.full((16,), seg_base, jnp.int32)

        def zbody(i, _):
            zi = jnp.full((16,), 0, jnp.int32) + i
            for j in range(NVEC):
                plsc.store_scatter(table, [zi, iotas[j]], zero)
            return 0

        lax.fori_loop(0, LAST, zbody, 0)

        pltpu.sync_copy(offs_hbm, offs_v)
        r0 = offs_v[pl.ds(w, 16)][0]
        r1 = offs_v[pl.ds(w + 1, 16)][0]
        r0a = (r0 // 8) * 8
        nchunks = (r1 - r0a + C - 1) // C

        def chunk_base(c):
            return jnp.minimum(r0a + c * C, N - C)

        def start(c, fb, ib, semf, semi):
            base = chunk_base(c)
            pltpu.async_copy(feats_hbm.at[pl.ds(base * D, C * D)], fb, semf)
            pltpu.async_copy(batch_hbm.at[pl.ds(base, C)], ib.at[pl.ds(0, C)], semi)

        def drain(fb, ib, semf, semi):
            pltpu.make_async_copy(feats_hbm.at[pl.ds(0, C * D)], fb, semf).wait()
            pltpu.make_async_copy(batch_hbm.at[pl.ds(0, C)], ib.at[pl.ds(0, C)], semi).wait()

        def process(c, fb, ib, carry):
            base_u = r0a + c * C
            base = chunk_base(c)
            i0 = jnp.maximum(r0, jnp.minimum(base_u, r1)) - base
            i1 = jnp.minimum(r1, base_u + C) - base

            def ids_at(i):
                cur = _bcast0(ib[pl.ds(i, 16)])
                return cur, cur - segb_vec

            def step(i, prev, prow, cur, accs):
                row = [fb[pl.ds(i * D + j * 16, 16)] for j in range(NVEC)]
                bound = cur != prev
                for j in range(NVEC):
                    plsc.store_scatter(table, [prow, iotas[j]], accs[j],
                                       mask=bound)
                new_accs = tuple(
                    jnp.where(bound, row[j], jnp.maximum(accs[j], row[j]))
                    for j in range(NVEC)
                )
                return new_accs

            cur0, row0 = ids_at(i0)

            def body(g, lc):
                prev, prow, cur_p, row_p = lc[0], lc[1], lc[2], lc[3]
                accs = lc[4:]
                a = i0 + 4 * g
                accs = step(a, prev, prow, cur_p, accs)
                for u in range(1, 4):
                    i_u = jnp.minimum(a + u, i1 - 1)
                    cur_u, row_u = ids_at(i_u)
                    accs = step(i_u, cur_p, row_p, cur_u, accs)
                    cur_p, row_p = cur_u, row_u
                cur_n, row_n = ids_at(jnp.minimum(a + 4, i1 - 1))
                return (cur_p, row_p, cur_n, row_n) + accs

            out = lax.fori_loop(0, (i1 - i0 + 3) // 4, body,
                                (carry[0], carry[1], cur0, row0)
                                + tuple(carry[2:]))
            nonempty = i0 < i1
            nprev = jnp.where(nonempty, out[0], carry[0])
            nprow = jnp.where(nonempty, out[1], carry[1])
            return (nprev, nprow) + tuple(out[4:])

        start(jnp.int32(0), fbuf0, ibuf0, semf0, semi0)

        def pair_body(g, carry):
            c0 = 2 * g
            start(c0 + 1, fbuf1, ibuf1, semf1, semi1)
            drain(fbuf0, ibuf0, semf0, semi0)
            carry = process(c0, fbuf0, ibuf0, carry)
            start(c0 + 2, fbuf0, ibuf0, semf0, semi0)
            drain(fbuf1, ibuf1, semf1, semi1)
            return process(c0 + 1, fbuf1, ibuf1, carry)

        init = (jnp.full((16,), -1, jnp.int32),
                jnp.full((16,), LAST, jnp.int32)) + tuple(
            jnp.full((16,), -jnp.inf, jnp.float32) for _ in range(NVEC)
        )
        npairs = (nchunks + 1) // 2
        final = lax.fori_loop(0, npairs, pair_body, init)
        drain(fbuf0, ibuf0, semf0, semi0)
        for j in range(NVEC):
            plsc.store_scatter(table, [final[1], iotas[j]], final[2 + j])

        @pl.when(w < NW - 1)
        def _():
            pltpu.sync_copy(table.at[pl.ds(0, SPW)],
                            out_hbm.at[pl.ds(seg_base, SPW)])

        @pl.when(w == NW - 1)
        def _():
            pltpu.sync_copy(table.at[pl.ds(0, LAST)],
                            out_hbm.at[pl.ds(seg_base, LAST)])

    return seg_max


def kernel(feats, batch):
    N = feats.shape[0]
    bounds = jnp.concatenate(
        [jnp.arange(NW, dtype=jnp.int32) * SPW, jnp.full((1,), S, jnp.int32)]
    )
    offs = jnp.searchsorted(batch, bounds, side="left").astype(jnp.int32)
    offs = jnp.concatenate([offs, jnp.zeros((48 - NW - 1,), jnp.int32)])
    return _make_kernel(N)(feats.reshape(-1), batch, offs)
